# baseline (speedup 1.0000x reference)
import jax
import jax.numpy as jnp
from jax import lax
from jax.experimental import pallas as pl


def _pos_kernel(row_ref, col_ref, out_ref):
    H, W, D = 32, 32, 256
    col = col_ref[0:W, :]
    row = row_ref[0:H, :]
    lane = lax.broadcasted_iota(jnp.int32, (W, 128), 1)
    sel = lax.broadcasted_iota(jnp.int32, (W, 128), 0)
    dn = (((0,), (0,)), ((), ()))
    T = (lane % W == sel).astype(jnp.float32)
    x2 = lax.dot_general(col, T, dn, preferred_element_type=jnp.float32)
    x3 = jnp.broadcast_to(x2[:, None, :], (D, 8, 128))
    ys = []
    for s in range(8):
        Rs = (s * 4 + lane // W == sel).astype(jnp.float32)
        y2 = lax.dot_general(row, Rs, dn, preferred_element_type=jnp.float32)
        ys.append(y2[:, None, :])
    y3 = jnp.concatenate(ys, axis=1)
    out_ref[...] = jnp.concatenate([x3, y3], axis=0)


def kernel(row_embeddings, column_embeddings, x):
    batch, _, height, width = x.shape
    D = row_embeddings.shape[1]
    C = 2 * D
    out = pl.pallas_call(
        _pos_kernel,
        grid=(batch,),
        in_specs=[
            pl.BlockSpec(row_embeddings.shape, lambda b: (0, 0)),
            pl.BlockSpec(column_embeddings.shape, lambda b: (0, 0)),
        ],
        out_specs=pl.BlockSpec((C, 8, 128), lambda b: (b, 0, 0)),
        out_shape=jax.ShapeDtypeStruct((batch * C, 8, 128), jnp.float32),
    )(row_embeddings, column_embeddings)
    return out.reshape(batch, C, height, width)


# speedup vs baseline: 9.2412x; 9.2412x over previous
import jax
import jax.numpy as jnp
from jax import lax
from jax.experimental import pallas as pl


def _pos_kernel(row_ref, col_ref, out_ref):
    H, W, D = 32, 32, 256
    HW = H * W
    col = col_ref[0:W, :]
    row = row_ref[0:H, :]
    x_tile = jnp.concatenate([col] * H, axis=0)
    j = lax.broadcasted_iota(jnp.int32, (HW, H), 0)
    hsel = lax.broadcasted_iota(jnp.int32, (HW, H), 1)
    rep = (j // W == hsel).astype(jnp.float32)
    dn = (((1,), (0,)), ((), ()))
    y_tile = lax.dot_general(rep, row, dn, preferred_element_type=jnp.float32)
    out_ref[...] = jnp.concatenate([x_tile, y_tile], axis=1)


def kernel(row_embeddings, column_embeddings, x):
    batch, _, height, width = x.shape
    D = row_embeddings.shape[1]
    C = 2 * D
    HW = height * width
    out = pl.pallas_call(
        _pos_kernel,
        grid=(batch,),
        in_specs=[
            pl.BlockSpec(row_embeddings.shape, lambda b: (0, 0)),
            pl.BlockSpec(column_embeddings.shape, lambda b: (0, 0)),
        ],
        out_specs=pl.BlockSpec((HW, C), lambda b: (b, 0)),
        out_shape=jax.ShapeDtypeStruct((batch * HW, C), jnp.float32),
    )(row_embeddings, column_embeddings)
    return out.reshape(batch, height, width, C).transpose(0, 3, 1, 2)


# speedup vs baseline: 10.6093x; 1.1480x over previous
import jax
import jax.numpy as jnp
from jax import lax
from jax.experimental import pallas as pl
from jax.experimental.pallas import tpu as pltpu


def _pos_kernel(row_ref, col_ref, out_ref, scratch, sem):
    H, W, D = 32, 32, 256
    HW = H * W
    B = out_ref.shape[0]
    col = col_ref[0:W, :]
    row = row_ref[0:H, :]
    x_tile = jnp.concatenate([col] * H, axis=0)
    j = lax.broadcasted_iota(jnp.int32, (HW, H), 0)
    hsel = lax.broadcasted_iota(jnp.int32, (HW, H), 1)
    rep = (j // W == hsel).astype(jnp.float32)
    dn = (((1,), (0,)), ((), ()))
    y_tile = lax.dot_general(rep, row, dn, preferred_element_type=jnp.float32)
    scratch[...] = jnp.concatenate([x_tile, y_tile], axis=1)
    copies = [
        pltpu.make_async_copy(scratch, out_ref.at[b], sem.at[b]) for b in range(B)
    ]
    for c in copies:
        c.start()
    for c in copies:
        c.wait()


def kernel(row_embeddings, column_embeddings, x):
    batch, _, height, width = x.shape
    D = row_embeddings.shape[1]
    C = 2 * D
    HW = height * width
    out = pl.pallas_call(
        _pos_kernel,
        in_specs=[
            pl.BlockSpec(memory_space=pltpu.MemorySpace.VMEM),
            pl.BlockSpec(memory_space=pltpu.MemorySpace.VMEM),
        ],
        out_specs=pl.BlockSpec(memory_space=pltpu.MemorySpace.HBM),
        out_shape=jax.ShapeDtypeStruct((batch, HW, C), jnp.float32),
        scratch_shapes=[
            pltpu.VMEM((HW, C), jnp.float32),
            pltpu.SemaphoreType.DMA((batch,)),
        ],
    )(row_embeddings, column_embeddings)
    return out.reshape(batch, height, width, C).transpose(0, 3, 1, 2)
